# baseline (speedup 1.0000x reference)
import functools
import math

import jax
import jax.numpy as jnp
import numpy as np
from jax import lax
from jax.experimental import pallas as pl
from jax.experimental.pallas import tpu as pltpu
from jax.experimental.pallas import tpu_sc as plsc

N = 262144
D_FEAT = 128
M_SUB = 64
N_DIR = 16
N_CELLS = M_SUB * N_DIR
EPS = 1e-05

BM = 16384
NW = 32
BPW = N // NW
LANES = 16


_W_MAG = np.full((M_SUB, 8), float(N_DIR), np.float32)
_W_DIR = np.tile(np.arange(N_DIR, dtype=np.float32)[:, None], (1, 8))
_L_TRI = np.triu(np.ones((N_DIR, N_DIR), np.float32))


def _digitize_body(vec_ref, mean_ref, std_ref, cb_ref, dirs_ref,
                   wmag_ref, wdir_ref, ltri_ref, cell_ref):
    x = vec_ref[...]
    v = (x - mean_ref[...]) / (std_ref[...] + EPS)
    sq = jnp.sum(v * v, axis=1, keepdims=True)
    norm = jnp.sqrt(sq)
    ind_m = (norm > cb_ref[...]).astype(jnp.float32)
    m16 = jnp.dot(ind_m, wmag_ref[...],
                  preferred_element_type=jnp.float32)
    unit = v / (norm + 1e-12)
    raw = lax.dot_general(unit, dirs_ref[...], (((1,), (1,)), ((), ())),
                          preferred_element_type=jnp.float32)
    smax = jnp.max(raw, axis=1, keepdims=True)
    ind_d = raw == smax
    indf = ind_d.astype(jnp.float32)
    pref = jnp.dot(indf, ltri_ref[...],
                   preferred_element_type=jnp.float32)
    first = jnp.where(ind_d & (pref == 1.0), 1.0, 0.0)
    didx = jnp.dot(first, wdir_ref[...],
                   preferred_element_type=jnp.float32)
    cellf = m16 + didx
    cell_ref[...] = cellf[:, 0:1].astype(jnp.int32)


def _digitize(vectors, mean2, std2, cbounds, dirs_t):
    grid = (vectors.shape[0] // BM,)
    return pl.pallas_call(
        _digitize_body,
        grid=grid,
        in_specs=[
            pl.BlockSpec((BM, D_FEAT), lambda i: (i, 0)),
            pl.BlockSpec((1, D_FEAT), lambda i: (0, 0)),
            pl.BlockSpec((1, D_FEAT), lambda i: (0, 0)),
            pl.BlockSpec((1, M_SUB), lambda i: (0, 0)),
            pl.BlockSpec((N_DIR, D_FEAT), lambda i: (0, 0)),
            pl.BlockSpec((M_SUB, 8), lambda i: (0, 0)),
            pl.BlockSpec((N_DIR, 8), lambda i: (0, 0)),
            pl.BlockSpec((N_DIR, N_DIR), lambda i: (0, 0)),
        ],
        out_specs=pl.BlockSpec((BM, 1), lambda i: (i, 0)),
        out_shape=jax.ShapeDtypeStruct((vectors.shape[0], 1), jnp.int32),
        compiler_params=pltpu.CompilerParams(
            dimension_semantics=("parallel",)),
    )(vectors, mean2, std2, cbounds, dirs_t,
      jnp.asarray(_W_MAG), jnp.asarray(_W_DIR), jnp.asarray(_L_TRI))


def _tables_body(col_ref, grid_ref, mags_ref, prob_ref, rank_ref, cb_ref):
    c = col_ref[...]
    cum = jnp.zeros((N_CELLS, 1), jnp.int32)
    tot = jnp.int32(0)
    for i in range(N_CELLS // 128):
        row = grid_ref[i:i + 1, :]
        val = jnp.where(row > 0, row, 0)
        tot = tot + jnp.sum(val)
        hit = jnp.where(row <= c, jnp.broadcast_to(val, (N_CELLS, 128)), 0)
        cum = cum + jnp.sum(hit, axis=1, keepdims=True)
    totf = jnp.maximum(1, tot).astype(jnp.float32)
    cf = c.astype(jnp.float32)
    prob_ref[...] = cf / totf
    rank_ref[...] = (cum.astype(jnp.float32) - 0.5 * cf) / totf
    m = mags_ref[...]
    mid = (m[:, :M_SUB - 1] + m[:, 1:]) * jnp.float32(0.5)
    sc = mid * np.float32(math.sqrt(D_FEAT))
    cb_ref[...] = jnp.concatenate(
        [sc, jnp.full((1, 1), jnp.inf, jnp.float32)], axis=1)


def _tables(flat_col, flat_grid, mags2):
    return pl.pallas_call(
        _tables_body,
        out_shape=(jax.ShapeDtypeStruct((N_CELLS, 1), jnp.float32),
                   jax.ShapeDtypeStruct((N_CELLS, 1), jnp.float32),
                   jax.ShapeDtypeStruct((1, M_SUB), jnp.float32)),
    )(flat_col, flat_grid, mags2)


def _make_sc_gather_body(n_chunk):
    npw = n_chunk // NW

    def body_fn(cell_hbm, flat_hbm, prob_hbm, rank_hbm,
                bc_hbm, pr_hbm, rk_hbm,
                idx_v, flat_v, prob_v, rank_v, bc_v, pr_v, rk_v):
        wid = lax.axis_index("s") * 2 + lax.axis_index("c")
        base = wid * npw
        pltpu.sync_copy(cell_hbm.at[pl.ds(base, npw)], idx_v)
        pltpu.sync_copy(flat_hbm, flat_v)
        pltpu.sync_copy(prob_hbm, prob_v)
        pltpu.sync_copy(rank_hbm, rank_v)

        def body(i, carry):
            sl = pl.ds(i * LANES, LANES)
            idx = idx_v[sl]
            bc_v[sl] = plsc.load_gather(flat_v, [idx])
            pr_v[sl] = plsc.load_gather(prob_v, [idx])
            rk_v[sl] = plsc.load_gather(rank_v, [idx])
            return carry

        lax.fori_loop(0, npw // LANES, body, 0)
        pltpu.sync_copy(bc_v, bc_hbm.at[pl.ds(base, npw)])
        pltpu.sync_copy(pr_v, pr_hbm.at[pl.ds(base, npw)])
        pltpu.sync_copy(rk_v, rk_hbm.at[pl.ds(base, npw)])

    return body_fn


def _sc_gather(cell, flat, prob_t, rank_t):
    n_chunk = cell.shape[0]
    npw = n_chunk // NW
    mesh = plsc.VectorSubcoreMesh(core_axis_name="c", subcore_axis_name="s")
    fn = functools.partial(
        pl.kernel,
        mesh=mesh,
        out_type=(jax.ShapeDtypeStruct((n_chunk,), jnp.int32),
                  jax.ShapeDtypeStruct((n_chunk,), jnp.float32),
                  jax.ShapeDtypeStruct((n_chunk,), jnp.float32)),
        scratch_types=[
            pltpu.VMEM((npw,), jnp.int32),
            pltpu.VMEM((N_CELLS,), jnp.int32),
            pltpu.VMEM((N_CELLS,), jnp.float32),
            pltpu.VMEM((N_CELLS,), jnp.float32),
            pltpu.VMEM((npw,), jnp.int32),
            pltpu.VMEM((npw,), jnp.float32),
            pltpu.VMEM((npw,), jnp.float32),
        ],
        compiler_params=pltpu.CompilerParams(needs_layout_passes=False),
    )(_make_sc_gather_body(n_chunk))
    return fn(cell, flat, prob_t, rank_t)


def kernel(vectors, counts, mean, std, reference_magnitudes,
           reference_directions):
    flat = counts.reshape(-1)
    prob_t, rank_t, cb = _tables(flat.reshape(N_CELLS, 1),
                                 flat.reshape(N_CELLS // 128, 128),
                                 reference_magnitudes.reshape(1, M_SUB))
    mean2 = mean.reshape(1, D_FEAT)
    std2 = std.reshape(1, D_FEAT)
    prob1 = prob_t.reshape(N_CELLS)
    rank1 = rank_t.reshape(N_CELLS)
    half = N // 2
    outs = []
    for lo in (0, half):
        cell = _digitize(lax.slice_in_dim(vectors, lo, lo + half, axis=0),
                         mean2, std2, cb, reference_directions)
        outs.append(_sc_gather(cell.reshape(half), flat, prob1, rank1))
    bc = jnp.concatenate([outs[0][0], outs[1][0]])
    pr = jnp.concatenate([outs[0][1], outs[1][1]])
    rk = jnp.concatenate([outs[0][2], outs[1][2]])
    return bc, pr, rk


# speedup vs baseline: 1.3568x; 1.3568x over previous
import functools
import math

import jax
import jax.numpy as jnp
import numpy as np
from jax import lax
from jax.experimental import pallas as pl
from jax.experimental.pallas import tpu as pltpu
from jax.experimental.pallas import tpu_sc as plsc

N = 262144
D_FEAT = 128
M_SUB = 64
N_DIR = 16
N_CELLS = M_SUB * N_DIR
EPS = 1e-05

BM = 16384
NW = 32
BPW = N // NW
LANES = 16


_W_MAG = np.full((M_SUB, 8), float(N_DIR), np.float32)
_W_DIR = np.tile(np.arange(N_DIR, dtype=np.float32)[:, None], (1, 8))
_L_TRI = np.triu(np.ones((N_DIR, N_DIR), np.float32))


def _digitize_body(vec_ref, mean_ref, std_ref, cb_ref, dirs_ref,
                   wmag_ref, wdir_ref, ltri_ref, cell_ref):
    x = vec_ref[...]
    v = (x - mean_ref[...]) / (std_ref[...] + EPS)
    sq = jnp.sum(v * v, axis=1, keepdims=True)
    norm = jnp.sqrt(sq)
    ind_m = (norm > cb_ref[...]).astype(jnp.float32)
    m16 = jnp.dot(ind_m, wmag_ref[...],
                  preferred_element_type=jnp.float32)
    unit = v / (norm + 1e-12)
    raw = lax.dot_general(unit, dirs_ref[...], (((1,), (1,)), ((), ())),
                          preferred_element_type=jnp.float32)
    smax = jnp.max(raw, axis=1, keepdims=True)
    ind_d = raw == smax
    indf = ind_d.astype(jnp.float32)
    pref = jnp.dot(indf, ltri_ref[...],
                   preferred_element_type=jnp.float32)
    first = jnp.where(ind_d & (pref == 1.0), 1.0, 0.0)
    didx = jnp.dot(first, wdir_ref[...],
                   preferred_element_type=jnp.float32)
    cellf = m16 + didx
    cell_ref[...] = cellf[:, 0:1].astype(jnp.int32)


def _digitize(vectors, mean2, std2, cbounds, dirs_t, lo, nrows):
    grid = (nrows // BM,)
    off = lo // BM
    return pl.pallas_call(
        _digitize_body,
        grid=grid,
        in_specs=[
            pl.BlockSpec((BM, D_FEAT), lambda i: (i + off, 0)),
            pl.BlockSpec((1, D_FEAT), lambda i: (0, 0)),
            pl.BlockSpec((1, D_FEAT), lambda i: (0, 0)),
            pl.BlockSpec((1, M_SUB), lambda i: (0, 0)),
            pl.BlockSpec((N_DIR, D_FEAT), lambda i: (0, 0)),
            pl.BlockSpec((M_SUB, 8), lambda i: (0, 0)),
            pl.BlockSpec((N_DIR, 8), lambda i: (0, 0)),
            pl.BlockSpec((N_DIR, N_DIR), lambda i: (0, 0)),
        ],
        out_specs=pl.BlockSpec((BM, 1), lambda i: (i, 0)),
        out_shape=jax.ShapeDtypeStruct((nrows, 1), jnp.int32),
        compiler_params=pltpu.CompilerParams(
            dimension_semantics=("parallel",)),
    )(vectors, mean2, std2, cbounds, dirs_t,
      jnp.asarray(_W_MAG), jnp.asarray(_W_DIR), jnp.asarray(_L_TRI))


def _tables_body(col_ref, grid_ref, mags_ref, prob_ref, rank_ref, cb_ref):
    c = col_ref[...]
    cum = jnp.zeros((N_CELLS, 1), jnp.int32)
    tot = jnp.int32(0)
    for i in range(N_CELLS // 128):
        row = grid_ref[i:i + 1, :]
        val = jnp.where(row > 0, row, 0)
        tot = tot + jnp.sum(val)
        hit = jnp.where(row <= c, jnp.broadcast_to(val, (N_CELLS, 128)), 0)
        cum = cum + jnp.sum(hit, axis=1, keepdims=True)
    totf = jnp.maximum(1, tot).astype(jnp.float32)
    cf = c.astype(jnp.float32)
    prob_ref[...] = cf / totf
    rank_ref[...] = (cum.astype(jnp.float32) - 0.5 * cf) / totf
    m = mags_ref[...]
    mid = (m[:, :M_SUB - 1] + m[:, 1:]) * jnp.float32(0.5)
    sc = mid * np.float32(math.sqrt(D_FEAT))
    cb_ref[...] = jnp.concatenate(
        [sc, jnp.full((1, 1), jnp.inf, jnp.float32)], axis=1)


def _tables(flat_col, flat_grid, mags2):
    return pl.pallas_call(
        _tables_body,
        out_shape=(jax.ShapeDtypeStruct((N_CELLS, 1), jnp.float32),
                   jax.ShapeDtypeStruct((N_CELLS, 1), jnp.float32),
                   jax.ShapeDtypeStruct((1, M_SUB), jnp.float32)),
    )(flat_col, flat_grid, mags2)


def _make_sc_gather_body(n_chunk):
    npw = n_chunk // NW

    def body_fn(cell_hbm, flat_hbm, prob_hbm, rank_hbm,
                bc_hbm, pr_hbm, rk_hbm,
                idx_v, flat_v, prob_v, rank_v, bc_v, pr_v, rk_v):
        wid = lax.axis_index("s") * 2 + lax.axis_index("c")
        base = wid * npw
        pltpu.sync_copy(cell_hbm.at[pl.ds(base, npw)], idx_v)
        pltpu.sync_copy(flat_hbm, flat_v)
        pltpu.sync_copy(prob_hbm, prob_v)
        pltpu.sync_copy(rank_hbm, rank_v)

        def body(i, carry):
            sl = pl.ds(i * LANES, LANES)
            idx = idx_v[sl]
            bc_v[sl] = plsc.load_gather(flat_v, [idx])
            pr_v[sl] = plsc.load_gather(prob_v, [idx])
            rk_v[sl] = plsc.load_gather(rank_v, [idx])
            return carry

        lax.fori_loop(0, npw // LANES, body, 0)
        pltpu.sync_copy(bc_v, bc_hbm.at[pl.ds(base, npw)])
        pltpu.sync_copy(pr_v, pr_hbm.at[pl.ds(base, npw)])
        pltpu.sync_copy(rk_v, rk_hbm.at[pl.ds(base, npw)])

    return body_fn


def _sc_gather(cell, flat, prob_t, rank_t):
    n_chunk = cell.shape[0]
    npw = n_chunk // NW
    mesh = plsc.VectorSubcoreMesh(core_axis_name="c", subcore_axis_name="s")
    fn = functools.partial(
        pl.kernel,
        mesh=mesh,
        out_type=(jax.ShapeDtypeStruct((n_chunk,), jnp.int32),
                  jax.ShapeDtypeStruct((n_chunk,), jnp.float32),
                  jax.ShapeDtypeStruct((n_chunk,), jnp.float32)),
        scratch_types=[
            pltpu.VMEM((npw,), jnp.int32),
            pltpu.VMEM((N_CELLS,), jnp.int32),
            pltpu.VMEM((N_CELLS,), jnp.float32),
            pltpu.VMEM((N_CELLS,), jnp.float32),
            pltpu.VMEM((npw,), jnp.int32),
            pltpu.VMEM((npw,), jnp.float32),
            pltpu.VMEM((npw,), jnp.float32),
        ],
        compiler_params=pltpu.CompilerParams(needs_layout_passes=False),
    )(_make_sc_gather_body(n_chunk))
    return fn(cell, flat, prob_t, rank_t)


def kernel(vectors, counts, mean, std, reference_magnitudes,
           reference_directions):
    flat = counts.reshape(-1)
    prob_t, rank_t, cb = _tables(flat.reshape(N_CELLS, 1),
                                 flat.reshape(N_CELLS // 128, 128),
                                 reference_magnitudes.reshape(1, M_SUB))
    mean2 = mean.reshape(1, D_FEAT)
    std2 = std.reshape(1, D_FEAT)
    prob1 = prob_t.reshape(N_CELLS)
    rank1 = rank_t.reshape(N_CELLS)
    half = N // 2
    outs = []
    for lo in (0, half):
        cell = _digitize(vectors, mean2, std2, cb, reference_directions,
                         lo, half)
        outs.append(_sc_gather(cell.reshape(half), flat, prob1, rank1))
    bc = jnp.concatenate([outs[0][0], outs[1][0]])
    pr = jnp.concatenate([outs[0][1], outs[1][1]])
    rk = jnp.concatenate([outs[0][2], outs[1][2]])
    return bc, pr, rk


# speedup vs baseline: 1.4304x; 1.0543x over previous
import functools
import math

import jax
import jax.numpy as jnp
import numpy as np
from jax import lax
from jax.experimental import pallas as pl
from jax.experimental.pallas import tpu as pltpu
from jax.experimental.pallas import tpu_sc as plsc

N = 262144
D_FEAT = 128
M_SUB = 64
N_DIR = 16
N_CELLS = M_SUB * N_DIR
EPS = 1e-05

BM = 16384
NW = 32
BPW = N // NW
LANES = 16


_W_MAG = np.full((M_SUB, 8), float(N_DIR), np.float32)
_W_DIR = np.tile(np.arange(N_DIR, dtype=np.float32)[:, None], (1, 8))
_L_TRI = np.triu(np.ones((N_DIR, N_DIR), np.float32))


def _digitize_body(vec_ref, mean_ref, std_ref, mags_ref, dirs_ref,
                   wmag_ref, wdir_ref, ltri_ref, col_ref, grid_ref,
                   cell_ref, prob_ref, rank_ref):
    mg = mags_ref[...]
    mid = (mg[:, :M_SUB - 1] + mg[:, 1:]) * jnp.float32(0.5)
    sc = mid * np.float32(math.sqrt(D_FEAT))
    cb = jnp.concatenate(
        [sc, jnp.full((1, 1), jnp.inf, jnp.float32)], axis=1)

    @pl.when(pl.program_id(0) == 0)
    def _tables():
        c = col_ref[...]
        cum = jnp.zeros((N_CELLS, 1), jnp.int32)
        tot = jnp.int32(0)
        for i in range(N_CELLS // 128):
            row = grid_ref[i:i + 1, :]
            val = jnp.where(row > 0, row, 0)
            tot = tot + jnp.sum(val)
            hit = jnp.where(row <= c,
                            jnp.broadcast_to(val, (N_CELLS, 128)), 0)
            cum = cum + jnp.sum(hit, axis=1, keepdims=True)
        totf = jnp.maximum(1, tot).astype(jnp.float32)
        cf = c.astype(jnp.float32)
        prob_ref[...] = cf / totf
        rank_ref[...] = (cum.astype(jnp.float32) - 0.5 * cf) / totf

    x = vec_ref[...]
    v = (x - mean_ref[...]) / (std_ref[...] + EPS)
    sq = jnp.sum(v * v, axis=1, keepdims=True)
    norm = jnp.sqrt(sq)
    ind_m = (norm > cb).astype(jnp.float32)
    m16 = jnp.dot(ind_m, wmag_ref[...],
                  preferred_element_type=jnp.float32)
    unit = v / (norm + 1e-12)
    raw = lax.dot_general(unit, dirs_ref[...], (((1,), (1,)), ((), ())),
                          preferred_element_type=jnp.float32)
    smax = jnp.max(raw, axis=1, keepdims=True)
    ind_d = raw == smax
    indf = ind_d.astype(jnp.float32)
    pref = jnp.dot(indf, ltri_ref[...],
                   preferred_element_type=jnp.float32)
    first = jnp.where(ind_d & (pref == 1.0), 1.0, 0.0)
    didx = jnp.dot(first, wdir_ref[...],
                   preferred_element_type=jnp.float32)
    cellf = m16 + didx
    cell_ref[...] = cellf[:, 0:1].astype(jnp.int32)


def _digitize(vectors, mean2, std2, mags2, dirs, flat_col, flat_grid):
    grid = (N // BM,)
    return pl.pallas_call(
        _digitize_body,
        grid=grid,
        in_specs=[
            pl.BlockSpec((BM, D_FEAT), lambda i: (i, 0)),
            pl.BlockSpec((1, D_FEAT), lambda i: (0, 0)),
            pl.BlockSpec((1, D_FEAT), lambda i: (0, 0)),
            pl.BlockSpec((1, M_SUB), lambda i: (0, 0)),
            pl.BlockSpec((N_DIR, D_FEAT), lambda i: (0, 0)),
            pl.BlockSpec((M_SUB, 8), lambda i: (0, 0)),
            pl.BlockSpec((N_DIR, 8), lambda i: (0, 0)),
            pl.BlockSpec((N_DIR, N_DIR), lambda i: (0, 0)),
            pl.BlockSpec((N_CELLS, 1), lambda i: (0, 0)),
            pl.BlockSpec((N_CELLS // 128, 128), lambda i: (0, 0)),
        ],
        out_specs=(pl.BlockSpec((BM, 1), lambda i: (i, 0)),
                   pl.BlockSpec((N_CELLS, 1), lambda i: (0, 0)),
                   pl.BlockSpec((N_CELLS, 1), lambda i: (0, 0))),
        out_shape=(jax.ShapeDtypeStruct((N, 1), jnp.int32),
                   jax.ShapeDtypeStruct((N_CELLS, 1), jnp.float32),
                   jax.ShapeDtypeStruct((N_CELLS, 1), jnp.float32)),
        compiler_params=pltpu.CompilerParams(
            dimension_semantics=("arbitrary",)),
    )(vectors, mean2, std2, mags2, dirs,
      jnp.asarray(_W_MAG), jnp.asarray(_W_DIR), jnp.asarray(_L_TRI),
      flat_col, flat_grid)


def _sc_gather_body(cell_hbm, flat_hbm, prob_hbm, rank_hbm,
                    bc_hbm, pr_hbm, rk_hbm,
                    idx_v, flat_v, prob_v, rank_v, bc_v, pr_v, rk_v):
    wid = lax.axis_index("s") * 2 + lax.axis_index("c")
    base = wid * BPW
    pltpu.sync_copy(cell_hbm.at[pl.ds(base, BPW)], idx_v)
    pltpu.sync_copy(flat_hbm, flat_v)
    pltpu.sync_copy(prob_hbm, prob_v)
    pltpu.sync_copy(rank_hbm, rank_v)

    def body(i, carry):
        sl = pl.ds(i * LANES, LANES)
        idx = idx_v[sl]
        bc_v[sl] = plsc.load_gather(flat_v, [idx])
        pr_v[sl] = plsc.load_gather(prob_v, [idx])
        rk_v[sl] = plsc.load_gather(rank_v, [idx])
        return carry

    lax.fori_loop(0, BPW // LANES, body, 0)
    pltpu.sync_copy(bc_v, bc_hbm.at[pl.ds(base, BPW)])
    pltpu.sync_copy(pr_v, pr_hbm.at[pl.ds(base, BPW)])
    pltpu.sync_copy(rk_v, rk_hbm.at[pl.ds(base, BPW)])


def _sc_gather(cell, flat, prob_t, rank_t):
    mesh = plsc.VectorSubcoreMesh(core_axis_name="c", subcore_axis_name="s")
    fn = functools.partial(
        pl.kernel,
        mesh=mesh,
        out_type=(jax.ShapeDtypeStruct((N,), jnp.int32),
                  jax.ShapeDtypeStruct((N,), jnp.float32),
                  jax.ShapeDtypeStruct((N,), jnp.float32)),
        scratch_types=[
            pltpu.VMEM((BPW,), jnp.int32),
            pltpu.VMEM((N_CELLS,), jnp.int32),
            pltpu.VMEM((N_CELLS,), jnp.float32),
            pltpu.VMEM((N_CELLS,), jnp.float32),
            pltpu.VMEM((BPW,), jnp.int32),
            pltpu.VMEM((BPW,), jnp.float32),
            pltpu.VMEM((BPW,), jnp.float32),
        ],
        compiler_params=pltpu.CompilerParams(needs_layout_passes=False),
    )(_sc_gather_body)
    return fn(cell, flat, prob_t, rank_t)


def kernel(vectors, counts, mean, std, reference_magnitudes,
           reference_directions):
    flat = counts.reshape(-1)
    cell, prob_t, rank_t = _digitize(vectors,
                                     mean.reshape(1, D_FEAT),
                                     std.reshape(1, D_FEAT),
                                     reference_magnitudes.reshape(1, M_SUB),
                                     reference_directions,
                                     flat.reshape(N_CELLS, 1),
                                     flat.reshape(N_CELLS // 128, 128))
    bc, pr, rk = _sc_gather(cell.reshape(N), flat,
                            prob_t.reshape(N_CELLS), rank_t.reshape(N_CELLS))
    return bc, pr, rk
